# baseline (speedup 1.0000x reference)
import functools

import jax
import jax.numpy as jnp
import numpy as np
from jax import lax
from jax.experimental import pallas as pl
from jax.experimental.pallas import tpu as pltpu
from jax.experimental.pallas import tpu_sc as plsc

_BATCH = 16384
_CLS = 1000
_NUM_CLEAN = 64
_FORGET = 0.2
_GRADUAL = 10
_EPOCHS = 100


def _sched():
    rs = np.ones(_EPOCHS) * _FORGET
    rs[:_GRADUAL] = np.linspace(0.0, _FORGET, _GRADUAL)
    return rs


_K = int((1.0 - _sched()[5]) * _BATCH)

_S_TC = 8192
_S_SC = _BATCH - _S_TC
_S_SC_HALF = _S_SC // 2


_R = 1024


def _ce_body(x1_ref, x2_ref, lab_ref, l1_ref, l2_ref):
    lab = lab_ref[...]
    col = lax.broadcasted_iota(jnp.int32, (_R, _CLS), 1)
    onehot = col == lab
    for x_ref, out_ref in ((x1_ref, l1_ref), (x2_ref, l2_ref)):
        x = x_ref[...]
        m = jnp.max(x, axis=1, keepdims=True)
        s = jnp.sum(jnp.exp(x - m), axis=1, keepdims=True)
        xl = jnp.sum(jnp.where(onehot, x, 0.0), axis=1, keepdims=True)
        out_ref[...] = (m + jnp.log(s)) - xl


def _ce_losses_tc(logits, logits2, labels2d):
    grid = _S_TC // _R
    return pl.pallas_call(
        _ce_body,
        grid=(grid,),
        in_specs=[
            pl.BlockSpec((_R, _CLS), lambda i: (i, 0)),
            pl.BlockSpec((_R, _CLS), lambda i: (i, 0)),
            pl.BlockSpec((_R, 1), lambda i: (i, 0)),
        ],
        out_specs=[
            pl.BlockSpec((_R, 1), lambda i: (i, 0)),
            pl.BlockSpec((_R, 1), lambda i: (i, 0)),
        ],
        out_shape=[
            jax.ShapeDtypeStruct((_S_TC, 1), jnp.float32),
            jax.ShapeDtypeStruct((_S_TC, 1), jnp.float32),
        ],
    )(logits, logits2, labels2d)


_NW = 16
_RPW = _S_SC_HALF // _NW
_CHUNK = 32
_NCHUNK = _RPW // _CHUNK
_FULL = _CLS // 16
_TAIL = _CLS - _FULL * 16


def _sc_ce_body(row0, x1_hbm, x2_hbm, lab_hbm, s1_hbm, xl1_hbm, s2_hbm,
                xl2_hbm, buf0, buf1, lab_buf, s_scr, xl_scr, sem0, sem1):
    wid = lax.axis_index("s")
    base = wid * _RPW
    lane = lax.iota(jnp.int32, 16)
    pltpu.sync_copy(lab_hbm.at[pl.ds(base, _RPW)], lab_buf)

    def compute_chunk(ch, buf, s_hbm, xl_hbm):
        def row_body(t, carry):
            lab_b = lab_buf[ch * _CHUNK + t, :]
            accs = [jnp.zeros((16,), jnp.float32) for _ in range(4)]
            xlacc = jnp.zeros((16,), jnp.float32)
            for j in range(_FULL):
                x = buf[t, pl.ds(16 * j, 16)]
                accs[j % 4] = accs[j % 4] + jnp.exp(x)
                xlacc = jnp.where(lane + 16 * j == lab_b, x, xlacc)
            xt = buf[t, pl.ds(_CLS - 16, 16)]
            accs[3] = accs[3] + jnp.where(lane >= 16 - _TAIL,
                                          jnp.exp(xt), 0.0)
            xlacc = jnp.where(lane + (_CLS - 16) == lab_b, xt, xlacc)
            s_scr[t, :] = (accs[0] + accs[1]) + (accs[2] + accs[3])
            xl_scr[t, :] = xlacc
            return carry

        lax.fori_loop(0, _CHUNK, row_body, 0)
        pltpu.sync_copy(s_scr, s_hbm.at[pl.ds(base + ch * _CHUNK, _CHUNK), :])
        pltpu.sync_copy(xl_scr, xl_hbm.at[pl.ds(base + ch * _CHUNK, _CHUNK), :])

    def start(x_hbm, ch, buf, sem):
        r0 = row0 + base + ch * _CHUNK
        pltpu.async_copy(x_hbm.at[pl.ds(r0, _CHUNK), :], buf, sem)

    def wait(x_hbm, ch, buf, sem):
        r0 = row0 + base + ch * _CHUNK
        pltpu.make_async_copy(x_hbm.at[pl.ds(r0, _CHUNK), :], buf, sem).wait()

    for x_hbm, s_hbm, xl_hbm in ((x1_hbm, s1_hbm, xl1_hbm),
                                 (x2_hbm, s2_hbm, xl2_hbm)):
        start(x_hbm, 0, buf0, sem0)

        def pair_body(o, carry, x_hbm=x_hbm, s_hbm=s_hbm, xl_hbm=xl_hbm):
            ch0 = 2 * o
            ch1 = 2 * o + 1
            start(x_hbm, ch1, buf1, sem1)
            wait(x_hbm, ch0, buf0, sem0)
            compute_chunk(ch0, buf0, s_hbm, xl_hbm)

            @pl.when(ch0 + 2 < _NCHUNK)
            def _():
                start(x_hbm, ch0 + 2, buf0, sem0)

            wait(x_hbm, ch1, buf1, sem1)
            compute_chunk(ch1, buf1, s_hbm, xl_hbm)
            return carry

        lax.fori_loop(0, _NCHUNK // 2, pair_body, 0)


def _make_sc_ce(row0):
    @functools.partial(
        pl.kernel,
        mesh=plsc.VectorSubcoreMesh(core_axis_name="c", subcore_axis_name="s",
                                    num_cores=1),
        out_type=[jax.ShapeDtypeStruct((_S_SC_HALF, 16), jnp.float32)] * 4,
        scratch_types=[
            pltpu.VMEM((_CHUNK, _CLS), jnp.float32),
            pltpu.VMEM((_CHUNK, _CLS), jnp.float32),
            pltpu.VMEM((_RPW, 16), jnp.int32),
            pltpu.VMEM((_CHUNK, 16), jnp.float32),
            pltpu.VMEM((_CHUNK, 16), jnp.float32),
            pltpu.SemaphoreType.DMA,
            pltpu.SemaphoreType.DMA,
        ],
    )
    def sc_ce(x1_hbm, x2_hbm, lab_hbm, s1_hbm, xl1_hbm, s2_hbm, xl2_hbm,
              buf0, buf1, lab_buf, s_scr, xl_scr, sem0, sem1):
        _sc_ce_body(row0, x1_hbm, x2_hbm, lab_hbm, s1_hbm, xl1_hbm, s2_hbm,
                    xl2_hbm, buf0, buf1, lab_buf, s_scr, xl_scr, sem0, sem1)

    return sc_ce


_sc_ce_a = _make_sc_ce(_S_TC)
_sc_ce_b = _make_sc_ce(_S_TC + _S_SC_HALF)


def _radix_select(bits, pos, k):
    shape = bits.shape
    sel0 = jnp.zeros(shape, dtype=jnp.int32)
    cand0 = jnp.ones(shape, dtype=jnp.int32)

    def step(src, nbits):
        def body(j, carry):
            sel, cand, r = carry
            b = nbits - 1 - j
            bit = jnp.bitwise_and(lax.shift_right_logical(src, b), 1)
            zero = cand & (bit ^ 1)
            c = jnp.sum(zero)
            take_zero = r <= c
            sel = jnp.where(take_zero, sel, sel | zero)
            cand = jnp.where(take_zero, zero, cand & bit)
            r = jnp.where(take_zero, r, r - c)
            return sel, cand, r

        return body

    carry = (sel0, cand0, jnp.int32(k))
    carry = lax.fori_loop(0, 32, step(bits, 32), carry)
    carry = lax.fori_loop(0, 14, step(pos, 14), carry)
    sel, cand, _ = carry
    return (sel | cand) == 1


def _sel_body(lt1_ref, lt2_ref, s1a_ref, xl1a_ref, s2a_ref, xl2a_ref,
              s1b_ref, xl1b_ref, s2b_ref, xl2b_ref, idx_ref,
              o1_ref, o2_ref):
    gcol = lax.broadcasted_iota(jnp.int32, (2048, 128), 0)
    grow = lax.broadcasted_iota(jnp.int32, (2048, 128), 1)
    gmat = jnp.where(gcol // 16 == grow, 1.0, 0.0)

    def reduce16(ref):
        return jax.lax.dot(ref[...], gmat,
                           preferred_element_type=jnp.float32)

    l1 = jnp.concatenate(
        [lt1_ref[...],
         jnp.log(reduce16(s1a_ref)) - reduce16(xl1a_ref),
         jnp.log(reduce16(s1b_ref)) - reduce16(xl1b_ref)], axis=0)
    l2 = jnp.concatenate(
        [lt2_ref[...],
         jnp.log(reduce16(s2a_ref)) - reduce16(xl2a_ref),
         jnp.log(reduce16(s2b_ref)) - reduce16(xl2b_ref)], axis=0)
    filt = idx_ref[...] < _NUM_CLEAN
    row = lax.broadcasted_iota(jnp.int32, l1.shape, 0)
    col = lax.broadcasted_iota(jnp.int32, l1.shape, 1)
    pos = row * l1.shape[1] + col
    sel1 = _radix_select(lax.bitcast_convert_type(l1, jnp.int32), pos, _K)
    sel2 = _radix_select(lax.bitcast_convert_type(l2, jnp.int32), pos, _K)
    o1_ref[...] = jnp.sum(jnp.where(sel2 & filt, l1, 0.0))[None, None]
    o2_ref[...] = jnp.sum(jnp.where(sel1 & filt, l2, 0.0))[None, None]


def _select_sums(args):
    return pl.pallas_call(
        _sel_body,
        out_shape=[
            jax.ShapeDtypeStruct((1, 1), jnp.float32),
            jax.ShapeDtypeStruct((1, 1), jnp.float32),
        ],
    )(*args)


def kernel(logits, logits2, labels, epoch, index):
    lab_a = jnp.broadcast_to(
        labels[_S_TC:_S_TC + _S_SC_HALF, None], (_S_SC_HALF, 16))
    lab_b = jnp.broadcast_to(
        labels[_S_TC + _S_SC_HALF:, None], (_S_SC_HALF, 16))
    parts_a = _sc_ce_a(logits, logits2, lab_a)
    parts_b = _sc_ce_b(logits, logits2, lab_b)
    labels2d = labels[:_S_TC].reshape(_S_TC, 1)
    lt1, lt2 = _ce_losses_tc(logits, logits2, labels2d)
    hp = _S_SC_HALF // 128
    o1, o2 = _select_sums(
        [lt1.reshape(_S_TC // 128, 128), lt2.reshape(_S_TC // 128, 128)]
        + [p.reshape(hp, 2048) for p in parts_a]
        + [p.reshape(hp, 2048) for p in parts_b]
        + [index.reshape(128, 128)]
    )
    rs = jnp.asarray(_sched(), dtype=jnp.float32)
    num_remember_t = jnp.floor((1.0 - rs[epoch]) * _BATCH)
    return (o1[0, 0] / num_remember_t, o2[0, 0] / num_remember_t)


# speedup vs baseline: 1.2903x; 1.2903x over previous
import functools

import jax
import jax.numpy as jnp
import numpy as np
from jax import lax
from jax.experimental import pallas as pl
from jax.experimental.pallas import tpu as pltpu
from jax.experimental.pallas import tpu_sc as plsc

_BATCH = 16384
_CLS = 1000
_NUM_CLEAN = 64
_FORGET = 0.2
_GRADUAL = 10
_EPOCHS = 100


def _sched():
    rs = np.ones(_EPOCHS) * _FORGET
    rs[:_GRADUAL] = np.linspace(0.0, _FORGET, _GRADUAL)
    return rs


_K = int((1.0 - _sched()[5]) * _BATCH)

_S_TC = 12288
_S_SC = _BATCH - _S_TC
_S_SC_HALF = _S_SC // 2


_R = 1024


def _ce_body(x1_ref, x2_ref, lab_ref, l1_ref, l2_ref):
    lab = lab_ref[...]
    col = lax.broadcasted_iota(jnp.int32, (_R, _CLS), 1)
    onehot = col == lab
    for x_ref, out_ref in ((x1_ref, l1_ref), (x2_ref, l2_ref)):
        x = x_ref[...]
        m = jnp.max(x, axis=1, keepdims=True)
        s = jnp.sum(jnp.exp(x - m), axis=1, keepdims=True)
        xl = jnp.sum(jnp.where(onehot, x, 0.0), axis=1, keepdims=True)
        out_ref[...] = (m + jnp.log(s)) - xl


def _ce_losses_tc(logits, logits2, labels2d):
    grid = _S_TC // _R
    return pl.pallas_call(
        _ce_body,
        grid=(grid,),
        in_specs=[
            pl.BlockSpec((_R, _CLS), lambda i: (i, 0)),
            pl.BlockSpec((_R, _CLS), lambda i: (i, 0)),
            pl.BlockSpec((_R, 1), lambda i: (i, 0)),
        ],
        out_specs=[
            pl.BlockSpec((_R, 1), lambda i: (i, 0)),
            pl.BlockSpec((_R, 1), lambda i: (i, 0)),
        ],
        out_shape=[
            jax.ShapeDtypeStruct((_S_TC, 1), jnp.float32),
            jax.ShapeDtypeStruct((_S_TC, 1), jnp.float32),
        ],
    )(logits, logits2, labels2d)


_NC = 2
_NS = 16
_NW = _NC * _NS
_RPW = _S_SC // _NW
_CHUNK = 32
_NCHUNK = _RPW // _CHUNK
_FULL = _CLS // 16
_TAIL = _CLS - _FULL * 16


def _sc_ce_body(row0, x1_hbm, x2_hbm, lab_hbm, s1_hbm, xl1_hbm, s2_hbm,
                xl2_hbm, buf0, buf1, lab_buf, s_scr, xl_scr, sem0, sem1):
    wid = lax.axis_index("s") * _NC + lax.axis_index("c")
    base = wid * _RPW
    lane = lax.iota(jnp.int32, 16)
    pltpu.sync_copy(lab_hbm.at[pl.ds(base, _RPW)], lab_buf)

    def compute_chunk(ch, buf, s_hbm, xl_hbm):
        def row_body(t, carry):
            lab_b = lab_buf[ch * _CHUNK + t, :]
            accs = [jnp.zeros((16,), jnp.float32) for _ in range(4)]
            xlacc = jnp.zeros((16,), jnp.float32)
            for j in range(_FULL):
                x = buf[t, pl.ds(16 * j, 16)]
                accs[j % 4] = accs[j % 4] + jnp.exp(x)
                xlacc = jnp.where(lane + 16 * j == lab_b, x, xlacc)
            xt = buf[t, pl.ds(_CLS - 16, 16)]
            accs[3] = accs[3] + jnp.where(lane >= 16 - _TAIL,
                                          jnp.exp(xt), 0.0)
            xlacc = jnp.where(lane + (_CLS - 16) == lab_b, xt, xlacc)
            s_scr[t, :] = (accs[0] + accs[1]) + (accs[2] + accs[3])
            xl_scr[t, :] = xlacc
            return carry

        lax.fori_loop(0, _CHUNK, row_body, 0)
        pltpu.sync_copy(s_scr, s_hbm.at[pl.ds(base + ch * _CHUNK, _CHUNK), :])
        pltpu.sync_copy(xl_scr, xl_hbm.at[pl.ds(base + ch * _CHUNK, _CHUNK), :])

    def start(x_hbm, ch, buf, sem):
        r0 = row0 + base + ch * _CHUNK
        pltpu.async_copy(x_hbm.at[pl.ds(r0, _CHUNK), :], buf, sem)

    def wait(x_hbm, ch, buf, sem):
        r0 = row0 + base + ch * _CHUNK
        pltpu.make_async_copy(x_hbm.at[pl.ds(r0, _CHUNK), :], buf, sem).wait()

    for x_hbm, s_hbm, xl_hbm in ((x1_hbm, s1_hbm, xl1_hbm),
                                 (x2_hbm, s2_hbm, xl2_hbm)):
        start(x_hbm, 0, buf0, sem0)

        def pair_body(o, carry, x_hbm=x_hbm, s_hbm=s_hbm, xl_hbm=xl_hbm):
            ch0 = 2 * o
            ch1 = 2 * o + 1
            start(x_hbm, ch1, buf1, sem1)
            wait(x_hbm, ch0, buf0, sem0)
            compute_chunk(ch0, buf0, s_hbm, xl_hbm)

            @pl.when(ch0 + 2 < _NCHUNK)
            def _():
                start(x_hbm, ch0 + 2, buf0, sem0)

            wait(x_hbm, ch1, buf1, sem1)
            compute_chunk(ch1, buf1, s_hbm, xl_hbm)
            return carry

        lax.fori_loop(0, _NCHUNK // 2, pair_body, 0)


def _make_sc_ce(row0):
    @functools.partial(
        pl.kernel,
        mesh=plsc.VectorSubcoreMesh(core_axis_name="c", subcore_axis_name="s"),
        out_type=[jax.ShapeDtypeStruct((_S_SC, 16), jnp.float32)] * 4,
        scratch_types=[
            pltpu.VMEM((_CHUNK, _CLS), jnp.float32),
            pltpu.VMEM((_CHUNK, _CLS), jnp.float32),
            pltpu.VMEM((_RPW, 16), jnp.int32),
            pltpu.VMEM((_CHUNK, 16), jnp.float32),
            pltpu.VMEM((_CHUNK, 16), jnp.float32),
            pltpu.SemaphoreType.DMA,
            pltpu.SemaphoreType.DMA,
        ],
    )
    def sc_ce(x1_hbm, x2_hbm, lab_hbm, s1_hbm, xl1_hbm, s2_hbm, xl2_hbm,
              buf0, buf1, lab_buf, s_scr, xl_scr, sem0, sem1):
        _sc_ce_body(row0, x1_hbm, x2_hbm, lab_hbm, s1_hbm, xl1_hbm, s2_hbm,
                    xl2_hbm, buf0, buf1, lab_buf, s_scr, xl_scr, sem0, sem1)

    return sc_ce


_sc_ce_one = _make_sc_ce(_S_TC)


def _radix_select(bits, pos, k):
    shape = bits.shape
    sel0 = jnp.zeros(shape, dtype=jnp.int32)
    cand0 = jnp.ones(shape, dtype=jnp.int32)

    def step(src, nbits):
        def body(j, carry):
            sel, cand, r = carry
            b = nbits - 1 - j
            bit = jnp.bitwise_and(lax.shift_right_logical(src, b), 1)
            zero = cand & (bit ^ 1)
            c = jnp.sum(zero)
            take_zero = r <= c
            sel = jnp.where(take_zero, sel, sel | zero)
            cand = jnp.where(take_zero, zero, cand & bit)
            r = jnp.where(take_zero, r, r - c)
            return sel, cand, r

        return body

    carry = (sel0, cand0, jnp.int32(k))
    carry = lax.fori_loop(0, 32, step(bits, 32), carry)
    carry = lax.fori_loop(0, 14, step(pos, 14), carry)
    sel, cand, _ = carry
    return (sel | cand) == 1


def _sel_body(lt1_ref, lt2_ref, s1_ref, xl1_ref, s2_ref, xl2_ref, idx_ref,
              o1_ref, o2_ref):
    gcol = lax.broadcasted_iota(jnp.int32, (2048, 128), 0)
    grow = lax.broadcasted_iota(jnp.int32, (2048, 128), 1)
    gmat = jnp.where(gcol // 16 == grow, 1.0, 0.0)

    def reduce16(ref):
        return jax.lax.dot(ref[...], gmat,
                           preferred_element_type=jnp.float32)

    l1 = jnp.concatenate(
        [lt1_ref[...], jnp.log(reduce16(s1_ref)) - reduce16(xl1_ref)], axis=0)
    l2 = jnp.concatenate(
        [lt2_ref[...], jnp.log(reduce16(s2_ref)) - reduce16(xl2_ref)], axis=0)
    filt = idx_ref[...] < _NUM_CLEAN
    row = lax.broadcasted_iota(jnp.int32, l1.shape, 0)
    col = lax.broadcasted_iota(jnp.int32, l1.shape, 1)
    pos = row * l1.shape[1] + col
    sel1 = _radix_select(lax.bitcast_convert_type(l1, jnp.int32), pos, _K)
    sel2 = _radix_select(lax.bitcast_convert_type(l2, jnp.int32), pos, _K)
    o1_ref[...] = jnp.sum(jnp.where(sel2 & filt, l1, 0.0))[None, None]
    o2_ref[...] = jnp.sum(jnp.where(sel1 & filt, l2, 0.0))[None, None]


def _select_sums(args):
    return pl.pallas_call(
        _sel_body,
        out_shape=[
            jax.ShapeDtypeStruct((1, 1), jnp.float32),
            jax.ShapeDtypeStruct((1, 1), jnp.float32),
        ],
    )(*args)


def kernel(logits, logits2, labels, epoch, index):
    lab_sc = jnp.broadcast_to(labels[_S_TC:, None], (_S_SC, 16))
    parts = _sc_ce_one(logits, logits2, lab_sc)
    labels2d = labels[:_S_TC].reshape(_S_TC, 1)
    lt1, lt2 = _ce_losses_tc(logits, logits2, labels2d)
    hp = _S_SC // 128
    o1, o2 = _select_sums(
        [lt1.reshape(_S_TC // 128, 128), lt2.reshape(_S_TC // 128, 128)]
        + [p.reshape(hp, 2048) for p in parts]
        + [index.reshape(128, 128)]
    )
    rs = jnp.asarray(_sched(), dtype=jnp.float32)
    num_remember_t = jnp.floor((1.0 - rs[epoch]) * _BATCH)
    return (o1[0, 0] / num_remember_t, o2[0, 0] / num_remember_t)


# speedup vs baseline: 1.4860x; 1.1517x over previous
import jax
import jax.numpy as jnp
import numpy as np
from jax.experimental import pallas as pl
from jax.experimental.pallas import tpu as pltpu

_BATCH = 16384
_CLS = 1000
_NUM_CLEAN = 64
_FORGET = 0.2
_GRADUAL = 10
_EPOCHS = 100


def _sched():
    rs = np.ones(_EPOCHS) * _FORGET
    rs[:_GRADUAL] = np.linspace(0.0, _FORGET, _GRADUAL)
    return rs


_K = int((1.0 - _sched()[5]) * _BATCH)

_R = 2048


def _ce_body(x1_ref, x2_ref, lab_ref, l1_ref, l2_ref):
    lab = lab_ref[...]
    col = jax.lax.broadcasted_iota(jnp.int32, (_R, _CLS), 1)
    onehot = col == lab
    for x_ref, out_ref in ((x1_ref, l1_ref), (x2_ref, l2_ref)):
        x = x_ref[...]
        m = jnp.max(x, axis=1, keepdims=True)
        s = jnp.sum(jnp.exp(x - m), axis=1, keepdims=True)
        xl = jnp.sum(jnp.where(onehot, x, 0.0), axis=1, keepdims=True)
        out_ref[...] = (m + jnp.log(s)) - xl


def _ce_losses(logits, logits2, labels2d):
    grid = _BATCH // _R
    return pl.pallas_call(
        _ce_body,
        grid=(grid,),
        in_specs=[
            pl.BlockSpec((_R, _CLS), lambda i: (i, 0)),
            pl.BlockSpec((_R, _CLS), lambda i: (i, 0)),
            pl.BlockSpec((_R, 1), lambda i: (i, 0)),
        ],
        out_specs=[
            pl.BlockSpec((_R, 1), lambda i: (i, 0)),
            pl.BlockSpec((_R, 1), lambda i: (i, 0)),
        ],
        out_shape=[
            jax.ShapeDtypeStruct((_BATCH, 1), jnp.float32),
            jax.ShapeDtypeStruct((_BATCH, 1), jnp.float32),
        ],
    )(logits, logits2, labels2d)


def _radix_select(bits, pos, k):
    shape = bits.shape
    sel0 = jnp.zeros(shape, dtype=jnp.int32)
    cand0 = jnp.ones(shape, dtype=jnp.int32)

    def step(src, nbits):
        def body(j, carry):
            sel, cand, r = carry
            b = nbits - 1 - j
            bit = jnp.bitwise_and(jax.lax.shift_right_logical(src, b), 1)
            zero = cand & (bit ^ 1)
            c = jnp.sum(zero)
            take_zero = r <= c
            sel = jnp.where(take_zero, sel, sel | zero)
            cand = jnp.where(take_zero, zero, cand & bit)
            r = jnp.where(take_zero, r, r - c)
            return sel, cand, r

        return body

    carry = (sel0, cand0, jnp.int32(k))
    carry = jax.lax.fori_loop(0, 32, step(bits, 32), carry)
    carry = jax.lax.fori_loop(0, 14, step(pos, 14), carry)
    sel, cand, _ = carry
    return (sel | cand) == 1


def _sel_body(l1_ref, l2_ref, idx_ref, s1_ref, s2_ref):
    l1 = l1_ref[...]
    l2 = l2_ref[...]
    filt = idx_ref[...] < _NUM_CLEAN
    row = jax.lax.broadcasted_iota(jnp.int32, l1.shape, 0)
    col = jax.lax.broadcasted_iota(jnp.int32, l1.shape, 1)
    pos = row * l1.shape[1] + col
    sel1 = _radix_select(jax.lax.bitcast_convert_type(l1, jnp.int32), pos, _K)
    sel2 = _radix_select(jax.lax.bitcast_convert_type(l2, jnp.int32), pos, _K)
    s1_ref[...] = jnp.sum(jnp.where(sel2 & filt, l1, 0.0))[None, None]
    s2_ref[...] = jnp.sum(jnp.where(sel1 & filt, l2, 0.0))[None, None]


def _select_sums(loss1, loss2, idx):
    return pl.pallas_call(
        _sel_body,
        out_shape=[
            jax.ShapeDtypeStruct((1, 1), jnp.float32),
            jax.ShapeDtypeStruct((1, 1), jnp.float32),
        ],
    )(loss1, loss2, idx)


def kernel(logits, logits2, labels, epoch, index):
    labels2d = labels.reshape(_BATCH, 1)
    loss1, loss2 = _ce_losses(logits, logits2, labels2d)
    s1, s2 = _select_sums(
        loss1.reshape(128, 128), loss2.reshape(128, 128), index.reshape(128, 128)
    )
    rs = jnp.asarray(_sched(), dtype=jnp.float32)
    num_remember_t = jnp.floor((1.0 - rs[epoch]) * _BATCH)
    return (s1[0, 0] / num_remember_t, s2[0, 0] / num_remember_t)


# speedup vs baseline: 1.5761x; 1.0606x over previous
import jax
import jax.numpy as jnp
import numpy as np
from jax import lax
from jax.experimental import pallas as pl
from jax.experimental.pallas import tpu as pltpu

_BATCH = 16384
_CLS = 1000
_NUM_CLEAN = 64
_FORGET = 0.2
_GRADUAL = 10
_EPOCHS = 100


def _sched():
    rs = np.ones(_EPOCHS) * _FORGET
    rs[:_GRADUAL] = np.linspace(0.0, _FORGET, _GRADUAL)
    return rs


_K = int((1.0 - _sched()[5]) * _BATCH)

_R = 2048
_GRID = _BATCH // _R
_RB = _R // 128


def _radix_select(bits, pos, k):
    shape = bits.shape
    sel0 = jnp.zeros(shape, dtype=jnp.int32)
    cand0 = jnp.ones(shape, dtype=jnp.int32)

    def step(src, nbits):
        def body(j, carry):
            sel, cand, r = carry
            b = nbits - 1 - j
            bit = jnp.bitwise_and(lax.shift_right_logical(src, b), 1)
            zero = cand & (bit ^ 1)
            c = jnp.sum(zero)
            take_zero = r <= c
            sel = jnp.where(take_zero, sel, sel | zero)
            cand = jnp.where(take_zero, zero, cand & bit)
            r = jnp.where(take_zero, r, r - c)
            return sel, cand, r

        return body

    carry = (sel0, cand0, jnp.int32(k))
    carry = lax.fori_loop(0, 32, step(bits, 32), carry)
    carry = lax.fori_loop(0, 14, step(pos, 14), carry)
    sel, cand, _ = carry
    return (sel | cand) == 1


def _fused_body(x1_ref, x2_ref, lab_ref, idx_ref, o1_ref, o2_ref,
                l1_scr, l2_scr):
    i = pl.program_id(0)
    col = lax.broadcasted_iota(jnp.int32, (_R, _CLS), 1)
    onehot = col == lab_ref[...]
    krow = lax.broadcasted_iota(jnp.int32, (_R, 128), 0)
    kcol = lax.broadcasted_iota(jnp.int32, (_R, 128), 1)
    wmask = (krow % 128) == kcol
    crow = lax.broadcasted_iota(jnp.int32, (_RB, _R), 0)
    ccol = lax.broadcasted_iota(jnp.int32, (_RB, _R), 1)
    cmat = jnp.where(ccol // 128 == crow, 1.0, 0.0)
    for x_ref, scr in ((x1_ref, l1_scr), (x2_ref, l2_scr)):
        x = x_ref[...]
        m = jnp.max(x, axis=1, keepdims=True)
        s = jnp.sum(jnp.exp(x - m), axis=1, keepdims=True)
        xl = jnp.sum(jnp.where(onehot, x, 0.0), axis=1, keepdims=True)
        loss = (m + jnp.log(s)) - xl
        w = jnp.where(wmask, loss, 0.0)
        blk = jax.lax.dot(cmat, w, preferred_element_type=jnp.float32)
        scr[pl.ds(i * _RB, _RB), :] = blk

    @pl.when(i == _GRID - 1)
    def _():
        l1 = l1_scr[...]
        l2 = l2_scr[...]
        filt = idx_ref[...] < _NUM_CLEAN
        row = lax.broadcasted_iota(jnp.int32, (128, 128), 0)
        colp = lax.broadcasted_iota(jnp.int32, (128, 128), 1)
        pos = row * 128 + colp
        sel1 = _radix_select(lax.bitcast_convert_type(l1, jnp.int32), pos, _K)
        sel2 = _radix_select(lax.bitcast_convert_type(l2, jnp.int32), pos, _K)
        o1_ref[...] = jnp.sum(jnp.where(sel2 & filt, l1, 0.0))[None, None]
        o2_ref[...] = jnp.sum(jnp.where(sel1 & filt, l2, 0.0))[None, None]


def _fused(logits, logits2, labels2d, idx):
    return pl.pallas_call(
        _fused_body,
        grid=(_GRID,),
        in_specs=[
            pl.BlockSpec((_R, _CLS), lambda i: (i, 0)),
            pl.BlockSpec((_R, _CLS), lambda i: (i, 0)),
            pl.BlockSpec((_R, 1), lambda i: (i, 0)),
            pl.BlockSpec((128, 128), lambda i: (0, 0)),
        ],
        out_specs=[
            pl.BlockSpec((1, 1), lambda i: (0, 0)),
            pl.BlockSpec((1, 1), lambda i: (0, 0)),
        ],
        out_shape=[
            jax.ShapeDtypeStruct((1, 1), jnp.float32),
            jax.ShapeDtypeStruct((1, 1), jnp.float32),
        ],
        scratch_shapes=[
            pltpu.VMEM((128, 128), jnp.float32),
            pltpu.VMEM((128, 128), jnp.float32),
        ],
    )(logits, logits2, labels2d, idx)


def kernel(logits, logits2, labels, epoch, index):
    o1, o2 = _fused(
        logits, logits2, labels.reshape(_BATCH, 1), index.reshape(128, 128)
    )
    rs = jnp.asarray(_sched(), dtype=jnp.float32)
    num_remember_t = jnp.floor((1.0 - rs[epoch]) * _BATCH)
    return (o1[0, 0] / num_remember_t, o2[0, 0] / num_remember_t)


# speedup vs baseline: 1.6219x; 1.0291x over previous
import jax
import jax.numpy as jnp
import numpy as np
from jax import lax
from jax.experimental import pallas as pl
from jax.experimental.pallas import tpu as pltpu

_BATCH = 16384
_CLS = 1000
_NUM_CLEAN = 64
_FORGET = 0.2
_GRADUAL = 10
_EPOCHS = 100


def _sched():
    rs = np.ones(_EPOCHS) * _FORGET
    rs[:_GRADUAL] = np.linspace(0.0, _FORGET, _GRADUAL)
    return rs


_K = int((1.0 - _sched()[5]) * _BATCH)

_R = 2048
_GRID = _BATCH // _R
_RB = _R // 128


def _radix_select(bits, pos, k):
    shape = bits.shape
    sel0 = jnp.zeros(shape, dtype=jnp.int32)
    cand0 = jnp.ones(shape, dtype=jnp.int32)

    def step(src, nbits):
        def body(j, carry):
            sel, cand, r = carry
            b = nbits - 1 - j
            bit = jnp.bitwise_and(lax.shift_right_logical(src, b), 1)
            zero = cand & (bit ^ 1)
            c = jnp.sum(zero)
            take_zero = r <= c
            sel = jnp.where(take_zero, sel, sel | zero)
            cand = jnp.where(take_zero, zero, cand & bit)
            r = jnp.where(take_zero, r, r - c)
            return sel, cand, r

        return body

    carry = (sel0, cand0, jnp.int32(k))
    carry = lax.fori_loop(0, 32, step(bits, 32), carry)
    carry = lax.fori_loop(0, 14, step(pos, 14), carry)
    sel, cand, _ = carry
    return (sel | cand) == 1


def _radix_select2(bits1, bits2, pos, k):
    shape = bits1.shape
    zero_m = jnp.zeros(shape, dtype=jnp.int32)
    one_m = jnp.ones(shape, dtype=jnp.int32)

    def step(src1, src2, nbits):
        def half(src, b, carry):
            sel, cand, r = carry
            bit = jnp.bitwise_and(lax.shift_right_logical(src, b), 1)
            zero = cand & (bit ^ 1)
            c = jnp.sum(zero)
            take_zero = r <= c
            sel = jnp.where(take_zero, sel, sel | zero)
            cand = jnp.where(take_zero, zero, cand & bit)
            r = jnp.where(take_zero, r, r - c)
            return sel, cand, r

        def body(j, carry):
            c1, c2 = carry
            b = nbits - 1 - j
            return half(src1, b, c1), half(src2, b, c2)

        return body

    carry = ((zero_m, one_m, jnp.int32(k)), (zero_m, one_m, jnp.int32(k)))
    carry = lax.fori_loop(0, 32, step(bits1, bits2, 32), carry)
    carry = lax.fori_loop(0, 14, step(pos, pos, 14), carry)
    (sel1, cand1, _), (sel2, cand2, _) = carry
    return (sel1 | cand1) == 1, (sel2 | cand2) == 1


def _fused_body(x1_ref, x2_ref, lab_ref, idx_ref, o1_ref, o2_ref,
                l1_scr, l2_scr):
    i = pl.program_id(0)
    col = lax.broadcasted_iota(jnp.int32, (_R, _CLS), 1)
    onehot = col == lab_ref[...]
    krow = lax.broadcasted_iota(jnp.int32, (_R, 128), 0)
    kcol = lax.broadcasted_iota(jnp.int32, (_R, 128), 1)
    wmask = (krow % 128) == kcol
    crow = lax.broadcasted_iota(jnp.int32, (_RB, _R), 0)
    ccol = lax.broadcasted_iota(jnp.int32, (_RB, _R), 1)
    cmat = jnp.where(ccol // 128 == crow, 1.0, 0.0)
    for x_ref, scr in ((x1_ref, l1_scr), (x2_ref, l2_scr)):
        x = x_ref[...]
        m = jnp.max(x, axis=1, keepdims=True)
        s = jnp.sum(jnp.exp(x - m), axis=1, keepdims=True)
        xl = jnp.sum(jnp.where(onehot, x, 0.0), axis=1, keepdims=True)
        loss = (m + jnp.log(s)) - xl
        w = jnp.where(wmask, loss, 0.0)
        blk = jax.lax.dot(cmat, w, preferred_element_type=jnp.float32)
        scr[pl.ds(i * _RB, _RB), :] = blk

    @pl.when(i == _GRID - 1)
    def _():
        l1 = l1_scr[...]
        l2 = l2_scr[...]
        filt = idx_ref[...] < _NUM_CLEAN
        row = lax.broadcasted_iota(jnp.int32, (128, 128), 0)
        colp = lax.broadcasted_iota(jnp.int32, (128, 128), 1)
        pos = row * 128 + colp
        sel1, sel2 = _radix_select2(
            lax.bitcast_convert_type(l1, jnp.int32),
            lax.bitcast_convert_type(l2, jnp.int32), pos, _K)
        o1_ref[...] = jnp.sum(jnp.where(sel2 & filt, l1, 0.0))[None, None]
        o2_ref[...] = jnp.sum(jnp.where(sel1 & filt, l2, 0.0))[None, None]


def _fused(logits, logits2, labels2d, idx):
    return pl.pallas_call(
        _fused_body,
        grid=(_GRID,),
        in_specs=[
            pl.BlockSpec((_R, _CLS), lambda i: (i, 0)),
            pl.BlockSpec((_R, _CLS), lambda i: (i, 0)),
            pl.BlockSpec((_R, 1), lambda i: (i, 0)),
            pl.BlockSpec((128, 128), lambda i: (0, 0)),
        ],
        out_specs=[
            pl.BlockSpec((1, 1), lambda i: (0, 0)),
            pl.BlockSpec((1, 1), lambda i: (0, 0)),
        ],
        out_shape=[
            jax.ShapeDtypeStruct((1, 1), jnp.float32),
            jax.ShapeDtypeStruct((1, 1), jnp.float32),
        ],
        scratch_shapes=[
            pltpu.VMEM((128, 128), jnp.float32),
            pltpu.VMEM((128, 128), jnp.float32),
        ],
    )(logits, logits2, labels2d, idx)


def kernel(logits, logits2, labels, epoch, index):
    o1, o2 = _fused(
        logits, logits2, labels.reshape(_BATCH, 1), index.reshape(128, 128)
    )
    rs = jnp.asarray(_sched(), dtype=jnp.float32)
    num_remember_t = jnp.floor((1.0 - rs[epoch]) * _BATCH)
    return (o1[0, 0] / num_remember_t, o2[0, 0] / num_remember_t)


# speedup vs baseline: 1.6229x; 1.0006x over previous
import jax
import jax.numpy as jnp
import numpy as np
from jax import lax
from jax.experimental import pallas as pl
from jax.experimental.pallas import tpu as pltpu

_BATCH = 16384
_CLS = 1000
_NUM_CLEAN = 64
_FORGET = 0.2
_GRADUAL = 10
_EPOCHS = 100


def _sched():
    rs = np.ones(_EPOCHS) * _FORGET
    rs[:_GRADUAL] = np.linspace(0.0, _FORGET, _GRADUAL)
    return rs


_K = int((1.0 - _sched()[5]) * _BATCH)

_R = 2048
_GRID = _BATCH // _R
_RB = _R // 128


def _radix_select2(bits1, bits2, pos, k):
    shape = bits1.shape
    zero_m = jnp.zeros(shape, dtype=jnp.int32)
    one_m = jnp.ones(shape, dtype=jnp.int32)

    def step(src1, src2, nbits):
        def half(src, b, carry):
            sel, cand, r = carry
            bit = jnp.bitwise_and(lax.shift_right_logical(src, b), 1)
            zero = cand & (bit ^ 1)
            c = jnp.sum(zero)
            take_zero = r <= c
            sel = jnp.where(take_zero, sel, sel | zero)
            cand = jnp.where(take_zero, zero, cand & bit)
            r = jnp.where(take_zero, r, r - c)
            return sel, cand, r

        def body(j, carry):
            c1, c2 = carry
            b = nbits - 1 - j
            return half(src1, b, c1), half(src2, b, c2)

        return body

    carry = ((zero_m, one_m, jnp.int32(k)), (zero_m, one_m, jnp.int32(k)))
    carry = lax.fori_loop(0, 32, step(bits1, bits2, 32), carry)
    carry = lax.fori_loop(0, 14, step(pos, pos, 14), carry)
    (sel1, cand1, _), (sel2, cand2, _) = carry
    return (sel1 | cand1) == 1, (sel2 | cand2) == 1


def _fused_body(x1_ref, x2_ref, lab_ref, idx_ref, o1_ref, o2_ref,
                l1_scr, l2_scr):
    i = pl.program_id(0)
    col = lax.broadcasted_iota(jnp.int32, (_R, _CLS), 1)
    onehot = col == lab_ref[...]
    krow = lax.broadcasted_iota(jnp.int32, (_R, 128), 0)
    kcol = lax.broadcasted_iota(jnp.int32, (_R, 128), 1)
    wmask = (krow % 128) == kcol
    crow = lax.broadcasted_iota(jnp.int32, (_RB, _R), 0)
    ccol = lax.broadcasted_iota(jnp.int32, (_RB, _R), 1)
    cmat = jnp.where(ccol // 128 == crow, 1.0, 0.0)
    for x_ref, scr in ((x1_ref, l1_scr), (x2_ref, l2_scr)):
        x = x_ref[...]
        m = jnp.max(x, axis=1, keepdims=True)
        s = jnp.sum(jnp.exp(x - m), axis=1, keepdims=True)
        xl = jnp.sum(jnp.where(onehot, x, 0.0), axis=1, keepdims=True)
        loss = (m + jnp.log(s)) - xl
        w = jnp.where(wmask, loss, 0.0)
        blk = jax.lax.dot(cmat, w, preferred_element_type=jnp.float32)
        scr[pl.ds(i * _RB, _RB), :] = blk

    @pl.when(i == _GRID - 1)
    def _():
        l1 = l1_scr[...]
        l2 = l2_scr[...]
        filt = idx_ref[...] < _NUM_CLEAN
        row = lax.broadcasted_iota(jnp.int32, (128, 128), 0)
        colp = lax.broadcasted_iota(jnp.int32, (128, 128), 1)
        pos = row * 128 + colp
        sel1, sel2 = _radix_select2(
            lax.bitcast_convert_type(l1, jnp.int32),
            lax.bitcast_convert_type(l2, jnp.int32), pos, _K)
        o1_ref[...] = jnp.sum(jnp.where(sel2 & filt, l1, 0.0))[None, None]
        o2_ref[...] = jnp.sum(jnp.where(sel1 & filt, l2, 0.0))[None, None]


def _fused(logits, logits2, labels2d, idx):
    return pl.pallas_call(
        _fused_body,
        grid=(_GRID,),
        in_specs=[
            pl.BlockSpec((_R, _CLS), lambda i: (i, 0)),
            pl.BlockSpec((_R, _CLS), lambda i: (i, 0)),
            pl.BlockSpec((_R, 1), lambda i: (i, 0)),
            pl.BlockSpec((128, 128), lambda i: (0, 0)),
        ],
        out_specs=[
            pl.BlockSpec((1, 1), lambda i: (0, 0)),
            pl.BlockSpec((1, 1), lambda i: (0, 0)),
        ],
        out_shape=[
            jax.ShapeDtypeStruct((1, 1), jnp.float32),
            jax.ShapeDtypeStruct((1, 1), jnp.float32),
        ],
        scratch_shapes=[
            pltpu.VMEM((128, 128), jnp.float32),
            pltpu.VMEM((128, 128), jnp.float32),
        ],
    )(logits, logits2, labels2d, idx)


def kernel(logits, logits2, labels, epoch, index):
    o1, o2 = _fused(
        logits, logits2, labels.reshape(_BATCH, 1), index.reshape(128, 128)
    )
    rs = jnp.asarray(_sched(), dtype=jnp.float32)
    num_remember_t = jnp.floor((1.0 - rs[epoch]) * _BATCH)
    return (o1[0, 0] / num_remember_t, o2[0, 0] / num_remember_t)
